# baseline (speedup 1.0000x reference)
import functools

import jax
import jax.numpy as jnp
from jax import lax
from jax.experimental import pallas as pl
from jax.experimental.pallas import tpu as pltpu
from jax.experimental.pallas import tpu_sc as plsc


def _pack_rows(x):
    h = x.shape[-1] // 2
    u = lax.bitcast_convert_type(x, jnp.uint32)
    rne = (u + jnp.uint32(0x7FFF) + ((u >> 16) & jnp.uint32(1))) >> 16
    return rne[:, :h] | (rne[:, h:] << 16)


def _premul_body(tok_ref, pat_ref, w_ref, t1_ref, p2_ref, t3_ref):
    d = tok_ref.shape[1]
    t = tok_ref[...]
    p = pat_ref[...]
    w = w_ref[...]
    t1_ref[...] = _pack_rows(jnp.dot(t, w[0:d, :],
                                     preferred_element_type=jnp.float32))
    p2_ref[...] = _pack_rows(jnp.dot(p, w[d:2 * d, :],
                                     preferred_element_type=jnp.float32))
    t3_ref[...] = _pack_rows(jnp.dot(t, w[2 * d:3 * d, :],
                                     preferred_element_type=jnp.float32))


def _premultiply(tokens_table, paths_table, W_t):
    n_tok, d = tokens_table.shape
    assert paths_table.shape[0] == n_tok
    h = d // 2
    rb = 2000
    grid = (n_tok // rb,)
    out_shape = [jax.ShapeDtypeStruct((n_tok, h), jnp.uint32)] * 3
    return pl.pallas_call(
        _premul_body,
        grid=grid,
        in_specs=[
            pl.BlockSpec((rb, d), lambda r: (r, 0)),
            pl.BlockSpec((rb, d), lambda r: (r, 0)),
            pl.BlockSpec((3 * d, d), lambda r: (0, 0)),
        ],
        out_specs=[pl.BlockSpec((rb, h), lambda r: (r, 0))] * 3,
        out_shape=out_shape,
    )(tokens_table, paths_table, W_t)


def _gather_sum(T1, P2, T3, i1, i2, i3):
    nr = i1.shape[0]
    h = T1.shape[1]
    info = plsc.get_sparse_core_info()
    nc, ns = info.num_cores, info.num_subcores
    nw = nc * ns
    chunk = 128
    per_w = nr // nw
    n_chunks = per_w // chunk
    assert per_w * nw == nr and n_chunks * chunk == per_w and n_chunks % 4 == 0

    @functools.partial(
        pl.kernel,
        mesh=plsc.VectorSubcoreMesh(core_axis_name="c", subcore_axis_name="s"),
        compiler_params=pltpu.CompilerParams(needs_layout_passes=False,
                                             use_tc_tiling_on_sc=False),
        out_type=jax.ShapeDtypeStruct((nr, 2 * h), jnp.float32),
        scratch_types=[
            pltpu.VMEM((2, 3, chunk), jnp.int32),
            pltpu.VMEM((2, 3, chunk, h), jnp.uint32),
            pltpu.VMEM((4, chunk, 2 * h), jnp.float32),
        ] + [pltpu.SemaphoreType.DMA] * 8,
    )
    def sc_kernel(i1_hbm, i2_hbm, i3_hbm, t1_hbm, p2_hbm, t3_hbm, out_hbm,
                  idx_v, rows_v, out_v,
                  isem0, isem1, gsem0, gsem1, osem0, osem1, osem2, osem3):
        isem = (isem0, isem1)
        gsem = (gsem0, gsem1)
        osem = (osem0, osem1, osem2, osem3)
        i_hbm = (i1_hbm, i2_hbm, i3_hbm)
        t_hbm = (t1_hbm, p2_hbm, t3_hbm)
        wid = lax.axis_index("s") * nc + lax.axis_index("c")
        base0 = wid * per_w

        def idx_src(c, j):
            return i_hbm[j].at[pl.ds(base0 + c * chunk, chunk)]

        def out_dst(c):
            return out_hbm.at[pl.ds(base0 + c * chunk, chunk)]

        def fire_gathers(b):
            for j in range(3):
                pltpu.async_copy(t_hbm[j].at[idx_v.at[b, j]], rows_v.at[b, j],
                                 gsem[b])

        def drain_gathers(b):
            for j in range(3):
                pltpu.make_async_copy(t_hbm[j].at[idx_v.at[b, j]],
                                      rows_v.at[b, j], gsem[b]).wait()

        def combine(b, o):
            def row(r, carry):
                for k in range(h // 16):
                    sl = pl.ds(k * 16, 16)
                    se = jnp.float32(0)
                    so = jnp.float32(0)
                    for j in range(3):
                        bf = plsc.bitcast(rows_v[b, j, r, sl], jnp.bfloat16)
                        e, od = plsc.unpack(bf,
                                            format=plsc.PackFormat.INTERLEAVED)
                        se = se + e
                        so = so + od
                    out_v[o, r, sl] = se
                    out_v[o, r, pl.ds(h + k * 16, 16)] = so
                return carry

            lax.fori_loop(0, chunk, row, 0)

        for c in (0, 1):
            for j in range(3):
                pltpu.sync_copy(idx_src(c, j), idx_v.at[c, j])
            fire_gathers(c)

        def group(g, carry):
            c0 = g * 4
            for u in range(4):
                b = u % 2
                o = u
                c = c0 + u
                drain_gathers(b)

                @pl.when(c + 2 < n_chunks)
                def _fire_idx():
                    for j in range(3):
                        pltpu.async_copy(idx_src(c + 2, j), idx_v.at[b, j],
                                         isem[b])

                @pl.when(c >= 4)
                def _wait_old_out():
                    pltpu.make_async_copy(out_v.at[o], out_dst(c - 4),
                                          osem[o]).wait()

                combine(b, o)
                pltpu.async_copy(out_v.at[o], out_dst(c), osem[o])

                @pl.when(c + 2 < n_chunks)
                def _prefetch():
                    for j in range(3):
                        pltpu.make_async_copy(idx_src(c + 2, j), idx_v.at[b, j],
                                              isem[b]).wait()
                    fire_gathers(b)

            return carry

        lax.fori_loop(0, n_chunks // 4, group, 0)

        for u in range(4):
            c = n_chunks - 4 + u
            pltpu.make_async_copy(out_v.at[u], out_dst(c), osem[u]).wait()

    return sc_kernel(i1, i2, i3, T1, P2, T3)


def _attend_body(s_ref, bt_ref, wa_ref, out_ref):
    t = jnp.tanh(s_ref[...] + bt_ref[...][None, None, :])
    logits = jnp.sum(t * wa_ref[...][None, None, :], axis=2)
    m = jnp.max(logits, axis=1, keepdims=True)
    e = jnp.exp(logits - m)
    attn = e / jnp.sum(e, axis=1, keepdims=True)
    out_ref[...] = jnp.sum(t * attn[:, :, None], axis=1)


def _attend(s, b_t, w_a):
    bsz, l, d = s.shape
    bb = 64
    grid = (bsz // bb,)
    vec = pl.BlockSpec((d,), lambda i: (0,))
    return pl.pallas_call(
        _attend_body,
        grid=grid,
        in_specs=[pl.BlockSpec((bb, l, d), lambda i: (i, 0, 0)), vec, vec],
        out_specs=pl.BlockSpec((bb, d), lambda i: (i, 0)),
        out_shape=jax.ShapeDtypeStruct((bsz, d), jnp.float32),
    )(s, b_t, w_a)


def kernel(contexts, tokens_table, paths_table, W_t, b_t, W_a, b_a):
    bsz, l = contexts.shape[1], contexts.shape[2]
    d = tokens_table.shape[1]
    h = d // 2
    perm = jnp.concatenate([jnp.arange(0, d, 2), jnp.arange(1, d, 2)])
    T1, P2, T3 = _premultiply(tokens_table, paths_table, W_t[:, perm])
    s = _gather_sum(T1, P2, T3, contexts[0].reshape(-1),
                    contexts[1].reshape(-1), contexts[2].reshape(-1))
    out_p = _attend(s.reshape(bsz, l, d), b_t[perm], W_a.reshape(-1)[perm])
    return out_p.reshape(bsz, 2, h).transpose(0, 2, 1).reshape(bsz, d)


# speedup vs baseline: 1.3195x; 1.3195x over previous
import functools

import jax
import jax.numpy as jnp
from jax import lax
from jax.experimental import pallas as pl
from jax.experimental.pallas import tpu as pltpu
from jax.experimental.pallas import tpu_sc as plsc


def _premul_body(tok_ref, pat_ref, w_ref, t1_ref, p2_ref, t3_ref):
    d = tok_ref.shape[1]
    t = tok_ref[...]
    p = pat_ref[...]
    w = w_ref[...]
    t1_ref[...] = jnp.dot(t, w[0:d, :], preferred_element_type=jnp.float32)
    p2_ref[...] = jnp.dot(p, w[d:2 * d, :], preferred_element_type=jnp.float32)
    t3_ref[...] = jnp.dot(t, w[2 * d:3 * d, :], preferred_element_type=jnp.float32)


def _premultiply(tokens_table, paths_table, W_t):
    n_tok, d = tokens_table.shape
    n_path = paths_table.shape[0]
    assert n_tok == n_path, "row-block premultiply assumes same table sizes"
    rb = 2000
    grid = (n_tok // rb,)
    out_shape = [jax.ShapeDtypeStruct((n_tok, d), jnp.float32)] * 3
    return pl.pallas_call(
        _premul_body,
        grid=grid,
        in_specs=[
            pl.BlockSpec((rb, d), lambda r: (r, 0)),
            pl.BlockSpec((rb, d), lambda r: (r, 0)),
            pl.BlockSpec((3 * d, d), lambda r: (0, 0)),
        ],
        out_specs=[pl.BlockSpec((rb, d), lambda r: (r, 0))] * 3,
        out_shape=out_shape,
    )(tokens_table, paths_table, W_t)


def _gather_sum(T1, P2, T3, i1, i2, i3):
    nr = i1.shape[0]
    d = T1.shape[1]
    info = plsc.get_sparse_core_info()
    nc, ns = info.num_cores, info.num_subcores
    nw = nc * ns
    chunk = 64
    per_w = nr // nw
    n_chunks = per_w // chunk
    assert per_w * nw == nr and n_chunks * chunk == per_w and n_chunks % 4 == 0

    @functools.partial(
        pl.kernel,
        mesh=plsc.VectorSubcoreMesh(core_axis_name="c", subcore_axis_name="s"),
        out_type=jax.ShapeDtypeStruct((nr, d), jnp.float32),
        scratch_types=[
            pltpu.VMEM((2, 3, chunk), jnp.int32),
            pltpu.VMEM((2, 2, chunk, d), jnp.float32),
            pltpu.VMEM((4, chunk, d), jnp.float32),
        ] + [pltpu.SemaphoreType.DMA] * 8,
    )
    def sc_kernel(i1_hbm, i2_hbm, i3_hbm, t1_hbm, p2_hbm, t3_hbm, out_hbm,
                  idx_v, rows_v, out_v,
                  isem0, isem1, gsem0, gsem1, osem0, osem1, osem2, osem3):
        isem = (isem0, isem1)
        gsem = (gsem0, gsem1)
        osem = (osem0, osem1, osem2, osem3)
        i_hbm = (i1_hbm, i2_hbm, i3_hbm)
        wid = lax.axis_index("s") * nc + lax.axis_index("c")
        base0 = wid * per_w

        def idx_src(c, j):
            return i_hbm[j].at[pl.ds(base0 + c * chunk, chunk)]

        def out_dst(c):
            return out_hbm.at[pl.ds(base0 + c * chunk, chunk)]

        def fire_side_gathers(b):
            pltpu.async_copy(p2_hbm.at[idx_v.at[b, 1]], rows_v.at[b, 0], gsem[b])
            pltpu.async_copy(t3_hbm.at[idx_v.at[b, 2]], rows_v.at[b, 1], gsem[b])

        def fire_main_gather(b, o):
            pltpu.async_copy(t1_hbm.at[idx_v.at[b, 0]], out_v.at[o], gsem[b])

        def drain_gathers(b, o):
            pltpu.make_async_copy(p2_hbm.at[idx_v.at[b, 1]], rows_v.at[b, 0],
                                  gsem[b]).wait()
            pltpu.make_async_copy(t3_hbm.at[idx_v.at[b, 2]], rows_v.at[b, 1],
                                  gsem[b]).wait()
            pltpu.make_async_copy(t1_hbm.at[idx_v.at[b, 0]], out_v.at[o],
                                  gsem[b]).wait()

        def combine(b, o):
            def row(r, carry):
                for k in range(d // 16):
                    sl = pl.ds(k * 16, 16)
                    plsc.addupdate(out_v.at[o, r, sl],
                                   rows_v[b, 0, r, sl] + rows_v[b, 1, r, sl])
                return carry

            lax.fori_loop(0, chunk, row, 0)

        for c in (0, 1):
            for j in range(3):
                pltpu.sync_copy(idx_src(c, j), idx_v.at[c, j])
            fire_side_gathers(c)
            fire_main_gather(c, c)

        def group(g, carry):
            c0 = g * 4
            for j in range(4):
                b = j % 2
                o = j
                o2 = (j + 2) % 4
                c = c0 + j
                drain_gathers(b, o)

                @pl.when(c + 2 < n_chunks)
                def _fire_idx():
                    for j in range(3):
                        pltpu.async_copy(idx_src(c + 2, j), idx_v.at[b, j],
                                         isem[b])

                combine(b, o)
                pltpu.async_copy(out_v.at[o], out_dst(c), osem[o])

                @pl.when(c + 2 < n_chunks)
                def _prefetch():
                    for j in range(3):
                        pltpu.make_async_copy(idx_src(c + 2, j), idx_v.at[b, j],
                                              isem[b]).wait()
                    fire_side_gathers(b)

                    @pl.when(c >= 2)
                    def _wait_old_out():
                        pltpu.make_async_copy(out_v.at[o2], out_dst(c - 2),
                                              osem[o2]).wait()

                    fire_main_gather(b, o2)

            return carry

        lax.fori_loop(0, n_chunks // 4, group, 0)

        for j in range(4):
            c = n_chunks - 4 + j
            pltpu.make_async_copy(out_v.at[j], out_dst(c), osem[j]).wait()

    return sc_kernel(i1, i2, i3, T1, P2, T3)


def _attend_body(s_ref, bt_ref, wa_ref, out_ref):
    t = jnp.tanh(s_ref[...] + bt_ref[...][None, None, :])
    logits = jnp.sum(t * wa_ref[...][None, None, :], axis=2)
    m = jnp.max(logits, axis=1, keepdims=True)
    e = jnp.exp(logits - m)
    attn = e / jnp.sum(e, axis=1, keepdims=True)
    out_ref[...] = jnp.sum(t * attn[:, :, None], axis=1)


def _attend(s, b_t, w_a):
    bsz, l, d = s.shape
    bb = 64
    grid = (bsz // bb,)
    return pl.pallas_call(
        _attend_body,
        grid=grid,
        in_specs=[
            pl.BlockSpec((bb, l, d), lambda i: (i, 0, 0)),
            pl.BlockSpec((d,), lambda i: (0,)),
            pl.BlockSpec((d,), lambda i: (0,)),
        ],
        out_specs=pl.BlockSpec((bb, d), lambda i: (i, 0)),
        out_shape=jax.ShapeDtypeStruct((bsz, d), jnp.float32),
    )(s, b_t, w_a)


def kernel(contexts, tokens_table, paths_table, W_t, b_t, W_a, b_a):
    bsz, l = contexts.shape[1], contexts.shape[2]
    d = tokens_table.shape[1]
    T1, P2, T3 = _premultiply(tokens_table, paths_table, W_t)
    s = _gather_sum(T1, P2, T3, contexts[0].reshape(-1),
                    contexts[1].reshape(-1), contexts[2].reshape(-1))
    return _attend(s.reshape(bsz, l, d), b_t, W_a.reshape(-1))


# speedup vs baseline: 1.3699x; 1.0382x over previous
import functools

import jax
import jax.numpy as jnp
from jax import lax
from jax.experimental import pallas as pl
from jax.experimental.pallas import tpu as pltpu
from jax.experimental.pallas import tpu_sc as plsc


def _premul_body(tok_ref, pat_ref, w_ref, t1_ref, p2_ref, t3_ref):
    d = tok_ref.shape[1]
    t = tok_ref[...]
    p = pat_ref[...]
    w = w_ref[...]
    t1_ref[...] = jnp.dot(t, w[0:d, :], preferred_element_type=jnp.float32)
    p2_ref[...] = jnp.dot(p, w[d:2 * d, :], preferred_element_type=jnp.float32)
    t3_ref[...] = jnp.dot(t, w[2 * d:3 * d, :], preferred_element_type=jnp.float32)


def _premultiply(tokens_table, paths_table, W_t):
    n_tok, d = tokens_table.shape
    n_path = paths_table.shape[0]
    assert n_tok == n_path, "row-block premultiply assumes same table sizes"
    rb = 2000
    grid = (n_tok // rb,)
    out_shape = [jax.ShapeDtypeStruct((n_tok, d), jnp.float32)] * 3
    return pl.pallas_call(
        _premul_body,
        grid=grid,
        in_specs=[
            pl.BlockSpec((rb, d), lambda r: (r, 0)),
            pl.BlockSpec((rb, d), lambda r: (r, 0)),
            pl.BlockSpec((3 * d, d), lambda r: (0, 0)),
        ],
        out_specs=[pl.BlockSpec((rb, d), lambda r: (r, 0))] * 3,
        out_shape=out_shape,
    )(tokens_table, paths_table, W_t)


def _gather_sum(T1, P2, T3, i1, i2, i3):
    nr = i1.shape[0]
    d = T1.shape[1]
    info = plsc.get_sparse_core_info()
    nc, ns = info.num_cores, info.num_subcores
    nw = nc * ns
    chunk = 128
    per_w = nr // nw
    n_chunks = per_w // chunk
    n_main = n_chunks - 2
    assert per_w * nw == nr and n_chunks * chunk == per_w and n_main % 6 == 0

    @functools.partial(
        pl.kernel,
        mesh=plsc.VectorSubcoreMesh(core_axis_name="c", subcore_axis_name="s"),
        out_type=jax.ShapeDtypeStruct((nr, d), jnp.float32),
        scratch_types=[
            pltpu.VMEM((2, 3, chunk), jnp.int32),
            pltpu.VMEM((2, 2, chunk, d), jnp.float32),
            pltpu.VMEM((3, chunk, d), jnp.float32),
        ] + [pltpu.SemaphoreType.DMA] * 7,
    )
    def sc_kernel(i1_hbm, i2_hbm, i3_hbm, t1_hbm, p2_hbm, t3_hbm, out_hbm,
                  idx_v, rows_v, acc_v,
                  isem0, isem1, gsem0, gsem1, osem0, osem1, osem2):
        isem = (isem0, isem1)
        gsem = (gsem0, gsem1)
        osem = (osem0, osem1, osem2)
        i_hbm = (i1_hbm, i2_hbm, i3_hbm)
        wid = lax.axis_index("s") * nc + lax.axis_index("c")
        base0 = wid * per_w

        def idx_src(c, j):
            return i_hbm[j].at[pl.ds(base0 + c * chunk, chunk)]

        def out_dst(c):
            return out_hbm.at[pl.ds(base0 + c * chunk, chunk)]

        def fire_side_gathers(b):
            pltpu.async_copy(p2_hbm.at[idx_v.at[b, 1]], rows_v.at[b, 0], gsem[b])
            pltpu.async_copy(t3_hbm.at[idx_v.at[b, 2]], rows_v.at[b, 1], gsem[b])

        def fire_main_gather(b, o):
            pltpu.async_copy(t1_hbm.at[idx_v.at[b, 0]], acc_v.at[o], gsem[b])

        def drain_gathers(b, o):
            pltpu.make_async_copy(p2_hbm.at[idx_v.at[b, 1]], rows_v.at[b, 0],
                                  gsem[b]).wait()
            pltpu.make_async_copy(t3_hbm.at[idx_v.at[b, 2]], rows_v.at[b, 1],
                                  gsem[b]).wait()
            pltpu.make_async_copy(t1_hbm.at[idx_v.at[b, 0]], acc_v.at[o],
                                  gsem[b]).wait()

        def combine(b, o):
            def row(r, carry):
                for k in range(d // 16):
                    sl = pl.ds(k * 16, 16)
                    plsc.addupdate(acc_v.at[o, r, sl],
                                   rows_v[b, 0, r, sl] + rows_v[b, 1, r, sl])
                return carry

            lax.fori_loop(0, chunk, row, 0)

        for c in (0, 1):
            for j in range(3):
                pltpu.sync_copy(idx_src(c, j), idx_v.at[c, j])
            fire_side_gathers(c)
            fire_main_gather(c, c)

        def group(g, carry):
            c0 = g * 6
            for u in range(6):
                b = u % 2
                o = u % 3
                o2 = (u + 2) % 3
                c = c0 + u
                drain_gathers(b, o)
                for j in range(3):
                    pltpu.async_copy(idx_src(c + 2, j), idx_v.at[b, j],
                                     isem[b])
                combine(b, o)
                pltpu.async_copy(acc_v.at[o], out_dst(c), osem[o])
                for j in range(3):
                    pltpu.make_async_copy(idx_src(c + 2, j), idx_v.at[b, j],
                                          isem[b]).wait()
                fire_side_gathers(b)

                @pl.when(c >= 1)
                def _wait_old_out():
                    pltpu.make_async_copy(acc_v.at[o2], out_dst(c - 1),
                                          osem[o2]).wait()

                fire_main_gather(b, o2)

            return carry

        lax.fori_loop(0, n_main // 6, group, 0)

        for c in (n_chunks - 2, n_chunks - 1):
            b = c % 2
            o = c % 3
            drain_gathers(b, o)
            combine(b, o)
            pltpu.async_copy(acc_v.at[o], out_dst(c), osem[o])
        for c in range(n_chunks - 3, n_chunks):
            pltpu.make_async_copy(acc_v.at[c % 3], out_dst(c),
                                  osem[c % 3]).wait()

    return sc_kernel(i1, i2, i3, T1, P2, T3)


def _attend_body(s_ref, bt_ref, wa_ref, out_ref):
    t = jnp.tanh(s_ref[...] + bt_ref[...][None, None, :])
    logits = jnp.sum(t * wa_ref[...][None, None, :], axis=2)
    m = jnp.max(logits, axis=1, keepdims=True)
    e = jnp.exp(logits - m)
    attn = e / jnp.sum(e, axis=1, keepdims=True)
    out_ref[...] = jnp.sum(t * attn[:, :, None], axis=1)


def _attend(s, b_t, w_a):
    bsz, l, d = s.shape
    bb = 64
    grid = (bsz // bb,)
    return pl.pallas_call(
        _attend_body,
        grid=grid,
        in_specs=[
            pl.BlockSpec((bb, l, d), lambda i: (i, 0, 0)),
            pl.BlockSpec((d,), lambda i: (0,)),
            pl.BlockSpec((d,), lambda i: (0,)),
        ],
        out_specs=pl.BlockSpec((bb, d), lambda i: (i, 0)),
        out_shape=jax.ShapeDtypeStruct((bsz, d), jnp.float32),
    )(s, b_t, w_a)


def kernel(contexts, tokens_table, paths_table, W_t, b_t, W_a, b_a):
    bsz, l = contexts.shape[1], contexts.shape[2]
    d = tokens_table.shape[1]
    T1, P2, T3 = _premultiply(tokens_table, paths_table, W_t)
    s = _gather_sum(T1, P2, T3, contexts[0].reshape(-1),
                    contexts[1].reshape(-1), contexts[2].reshape(-1))
    return _attend(s.reshape(bsz, l, d), b_t, W_a.reshape(-1))


# speedup vs baseline: 1.3866x; 1.0122x over previous
import functools

import jax
import jax.numpy as jnp
from jax import lax
from jax.experimental import pallas as pl
from jax.experimental.pallas import tpu as pltpu
from jax.experimental.pallas import tpu_sc as plsc


def _premul_body(tok_ref, pat_ref, w_ref, t1_ref, p2_ref, t3_ref):
    d = tok_ref.shape[1]
    t = tok_ref[...]
    p = pat_ref[...]
    w = w_ref[...]
    t1_ref[...] = jnp.dot(t, w[0:d, :], preferred_element_type=jnp.float32)
    p2_ref[...] = jnp.dot(p, w[d:2 * d, :], preferred_element_type=jnp.float32)
    t3_ref[...] = jnp.dot(t, w[2 * d:3 * d, :], preferred_element_type=jnp.float32)


def _premultiply(tokens_table, paths_table, W_t):
    n_tok, d = tokens_table.shape
    n_path = paths_table.shape[0]
    assert n_tok == n_path, "row-block premultiply assumes same table sizes"
    rb = 5000
    grid = (n_tok // rb,)
    out_shape = [jax.ShapeDtypeStruct((n_tok, d), jnp.float32)] * 3
    return pl.pallas_call(
        _premul_body,
        grid=grid,
        in_specs=[
            pl.BlockSpec((rb, d), lambda r: (r, 0)),
            pl.BlockSpec((rb, d), lambda r: (r, 0)),
            pl.BlockSpec((3 * d, d), lambda r: (0, 0)),
        ],
        out_specs=[pl.BlockSpec((rb, d), lambda r: (r, 0))] * 3,
        out_shape=out_shape,
    )(tokens_table, paths_table, W_t)


def _gather_sum(T1, P2, T3, i1, i2, i3):
    nr = i1.shape[0]
    d = T1.shape[1]
    info = plsc.get_sparse_core_info()
    nc, ns = info.num_cores, info.num_subcores
    nw = nc * ns
    chunk = 128
    per_w = nr // nw
    n_chunks = per_w // chunk
    n_main = n_chunks - 2
    assert per_w * nw == nr and n_chunks * chunk == per_w and n_main % 6 == 0

    @functools.partial(
        pl.kernel,
        mesh=plsc.VectorSubcoreMesh(core_axis_name="c", subcore_axis_name="s"),
        out_type=jax.ShapeDtypeStruct((nr, d), jnp.float32),
        scratch_types=[
            pltpu.VMEM((2, 3, chunk), jnp.int32),
            pltpu.VMEM((2, 2, chunk, d), jnp.float32),
            pltpu.VMEM((3, chunk, d), jnp.float32),
        ] + [pltpu.SemaphoreType.DMA] * 7,
    )
    def sc_kernel(i1_hbm, i2_hbm, i3_hbm, t1_hbm, p2_hbm, t3_hbm, out_hbm,
                  idx_v, rows_v, acc_v,
                  isem0, isem1, gsem0, gsem1, osem0, osem1, osem2):
        isem = (isem0, isem1)
        gsem = (gsem0, gsem1)
        osem = (osem0, osem1, osem2)
        i_hbm = (i1_hbm, i2_hbm, i3_hbm)
        wid = lax.axis_index("s") * nc + lax.axis_index("c")
        base0 = wid * per_w

        def idx_src(c, j):
            return i_hbm[j].at[pl.ds(base0 + c * chunk, chunk)]

        def out_dst(c):
            return out_hbm.at[pl.ds(base0 + c * chunk, chunk)]

        def fire_side_gathers(b):
            pltpu.async_copy(p2_hbm.at[idx_v.at[b, 1]], rows_v.at[b, 0], gsem[b])
            pltpu.async_copy(t3_hbm.at[idx_v.at[b, 2]], rows_v.at[b, 1], gsem[b])

        def fire_main_gather(b, o):
            pltpu.async_copy(t1_hbm.at[idx_v.at[b, 0]], acc_v.at[o], gsem[b])

        def drain_gathers(b, o):
            pltpu.make_async_copy(p2_hbm.at[idx_v.at[b, 1]], rows_v.at[b, 0],
                                  gsem[b]).wait()
            pltpu.make_async_copy(t3_hbm.at[idx_v.at[b, 2]], rows_v.at[b, 1],
                                  gsem[b]).wait()
            pltpu.make_async_copy(t1_hbm.at[idx_v.at[b, 0]], acc_v.at[o],
                                  gsem[b]).wait()

        def combine(b, o):
            def row(r, carry):
                for k in range(d // 16):
                    sl = pl.ds(k * 16, 16)
                    plsc.addupdate(acc_v.at[o, r, sl],
                                   rows_v[b, 0, r, sl] + rows_v[b, 1, r, sl])
                return carry

            lax.fori_loop(0, chunk, row, 0)

        for c in (0, 1):
            for j in range(3):
                pltpu.sync_copy(idx_src(c, j), idx_v.at[c, j])
            fire_side_gathers(c)
            fire_main_gather(c, c)

        def group(g, carry):
            c0 = g * 6
            for u in range(6):
                b = u % 2
                o = u % 3
                o2 = (u + 2) % 3
                c = c0 + u
                drain_gathers(b, o)
                for j in range(3):
                    pltpu.async_copy(idx_src(c + 2, j), idx_v.at[b, j],
                                     isem[b])
                combine(b, o)
                pltpu.async_copy(acc_v.at[o], out_dst(c), osem[o])
                for j in range(3):
                    pltpu.make_async_copy(idx_src(c + 2, j), idx_v.at[b, j],
                                          isem[b]).wait()
                fire_side_gathers(b)

                @pl.when(c >= 1)
                def _wait_old_out():
                    pltpu.make_async_copy(acc_v.at[o2], out_dst(c - 1),
                                          osem[o2]).wait()

                fire_main_gather(b, o2)

            return carry

        lax.fori_loop(0, n_main // 6, group, 0)

        for c in (n_chunks - 2, n_chunks - 1):
            b = c % 2
            o = c % 3
            drain_gathers(b, o)
            combine(b, o)
            pltpu.async_copy(acc_v.at[o], out_dst(c), osem[o])
        for c in range(n_chunks - 3, n_chunks):
            pltpu.make_async_copy(acc_v.at[c % 3], out_dst(c),
                                  osem[c % 3]).wait()

    return sc_kernel(i1, i2, i3, T1, P2, T3)


def _attend_body(s_ref, bt_ref, wa_ref, out_ref):
    bb, l, d = s_ref.shape
    t = jnp.tanh(s_ref[...] + bt_ref[...][None, None, :])
    logits = jnp.dot(t.reshape(bb * l, d), wa_ref[...],
                     preferred_element_type=jnp.float32).reshape(bb, l)
    m = jnp.max(logits, axis=1, keepdims=True)
    e = jnp.exp(logits - m)
    attn = e / jnp.sum(e, axis=1, keepdims=True)
    out_ref[...] = jnp.sum(t * attn[:, :, None], axis=1)


def _attend(s, b_t, w_a):
    bsz, l, d = s.shape
    bb = 128
    grid = (bsz // bb,)
    return pl.pallas_call(
        _attend_body,
        grid=grid,
        in_specs=[
            pl.BlockSpec((bb, l, d), lambda i: (i, 0, 0)),
            pl.BlockSpec((d,), lambda i: (0,)),
            pl.BlockSpec((d,), lambda i: (0,)),
        ],
        out_specs=pl.BlockSpec((bb, d), lambda i: (i, 0)),
        out_shape=jax.ShapeDtypeStruct((bsz, d), jnp.float32),
    )(s, b_t, w_a)


def kernel(contexts, tokens_table, paths_table, W_t, b_t, W_a, b_a):
    bsz, l = contexts.shape[1], contexts.shape[2]
    d = tokens_table.shape[1]
    T1, P2, T3 = _premultiply(tokens_table, paths_table, W_t)
    s = _gather_sum(T1, P2, T3, contexts[0].reshape(-1),
                    contexts[1].reshape(-1), contexts[2].reshape(-1))
    return _attend(s.reshape(bsz, l, d), b_t, W_a.reshape(-1))
